# baseline (speedup 1.0000x reference)
import jax
import jax.numpy as jnp
from jax import lax
from jax.experimental import pallas as pl


def _pairwise_sq(a, b):
    d2 = (jnp.sum(a * a, -1, keepdims=True)
          - 2.0 * jnp.einsum('bnd,bmd->bnm', a, b)
          + jnp.sum(b * b, -1)[:, None, :])
    return jnp.maximum(d2, 0.0)


def _branch(x, idx, W, b):
    C = x.shape[-1]
    Wt, Wb = W[:C], W[C:]
    y = x @ Wb
    c = x @ (Wt - Wb) + b
    nb = jax.vmap(lambda yy, ii: yy[ii])(y, idx)
    return jax.nn.relu(jnp.max(nb, axis=2) + c)


def _block(x, idx1, idx2, p, pre):
    h1 = _branch(x, idx1, p[pre + '_Wa'], p[pre + '_ba'])
    h2 = _branch(x, idx2, p[pre + '_Wb'], p[pre + '_bb'])
    return jax.nn.relu(jnp.concatenate([h1, h2], -1) @ p[pre + '_Wo'] + p[pre + '_bo'])


def _stn15(x, p):
    h = jax.nn.relu(x @ p['stn_W1'] + p['stn_b1'])
    h = jax.nn.relu(h @ p['stn_W2'] + p['stn_b2'])
    h = jax.nn.relu(h @ p['stn_W3'] + p['stn_b3'])
    g = jnp.max(h, axis=1)
    g = jax.nn.relu(g @ p['stn_W4'] + p['stn_b4'])
    g = jax.nn.relu(g @ p['stn_W5'] + p['stn_b5'])
    t = (g @ p['stn_W6'] + p['stn_b6']).reshape(-1, 15, 15) + jnp.eye(15, dtype=x.dtype)[None]
    return jnp.einsum('bnc,bcd->bnd', x, t)


def _res_block(x, p, pre):
    h = jax.nn.relu(x @ p[pre + '_W1'] + p[pre + '_b1'])
    h = h @ p[pre + '_W2'] + p[pre + '_b2']
    return jax.nn.relu(h + x @ p[pre + '_Wp'] + p[pre + '_bp'])


def kernel(x, pos, params):
    p = params
    cd = _pairwise_sq(pos, pos)
    I = lax.top_k(-cd, 512)[1]

    x = _stn15(x, p)

    def feat_idx(xx, lo, hi):
        J = lax.top_k(-_pairwise_sq(xx, xx), hi)[1]
        return J[..., lo:hi]

    h_b1 = _block(x, I[..., 0:32], feat_idx(x, 0, 12), p, 'b1')
    h_f1 = _block(x, I[..., 0:32], I[..., 0:240:20], p, 'f1')
    x1 = jnp.concatenate([h_b1, h_f1], -1)
    h_b2 = _block(x1, I[..., 16:48], feat_idx(x1, 6, 18), p, 'b2')
    h_f2 = _block(x1, I[..., 14:46], I[..., 6:360:36], p, 'f2')
    x2 = jnp.concatenate([h_b2, h_f2], -1)
    h_b3 = _block(x2, I[..., 16:48], feat_idx(x2, 6, 18), p, 'b3')
    h_f3 = _block(x2, I[..., 14:46], I[..., 6:540:54], p, 'f3')
    x3 = jnp.concatenate([h_b3, h_f3], -1)

    h = jnp.concatenate([x1, x2, x3], -1)
    h = jax.nn.relu(h @ p['mlp1_W'] + p['mlp1_b'])
    h = jax.nn.relu(h @ p['mlp2_W'] + p['mlp2_b'])
    w = jax.nn.sigmoid(jnp.max(h, axis=1) @ p['fi_W'] + p['fi_b'])
    h = h * w[:, None, :]
    h = _res_block(h, p, 'r1')
    h = _res_block(h, p, 'r2')
    return h @ p['out_W'] + p['out_b']


# speedup vs baseline: 2.9086x; 2.9086x over previous
import functools

import jax
import jax.numpy as jnp
from jax import lax
from jax.experimental import pallas as pl
from jax.experimental.pallas import tpu as pltpu
from jax.experimental.pallas import tpu_sc as plsc

_N = 2048
_NW = 32


@functools.partial(jax.jit, static_argnames=('k', 'hp', 'chunk'))
def _sc_gather_max(y, idx_flat, *, k, hp, chunk=16):
    nodes_w = _N // _NW
    mesh = plsc.VectorSubcoreMesh(core_axis_name="c", subcore_axis_name="s")

    @functools.partial(
        pl.kernel, mesh=mesh,
        out_type=jax.ShapeDtypeStruct((_N, hp), jnp.float32),
        scratch_types=[
            pltpu.VMEM((chunk * k,), jnp.int32),
            pltpu.VMEM((chunk * k, hp), jnp.float32),
            pltpu.VMEM((chunk, hp), jnp.float32),
            pltpu.SemaphoreType.DMA,
        ],
    )
    def body(y_hbm, idx_hbm, out_hbm, idx_v, rows_v, out_v, sem):
        wid = lax.axis_index("s") * 2 + lax.axis_index("c")
        base = wid * nodes_w

        def chunk_body(ci, carry):
            nb = base + ci * chunk
            pltpu.sync_copy(idx_hbm.at[pl.ds(nb * k, chunk * k)], idx_v)
            pltpu.async_copy(y_hbm.at[idx_v], rows_v, sem).wait()

            def node_body(n, carry2):
                def ch_body(c, carry3):
                    def red(j, acc):
                        return jnp.maximum(acc, rows_v[n * k + j, pl.ds(c * 16, 16)])
                    acc = rows_v[n * k, pl.ds(c * 16, 16)]
                    acc = lax.fori_loop(1, k, red, acc)
                    out_v[n, pl.ds(c * 16, 16)] = acc
                    return carry3
                return lax.fori_loop(0, hp // 16, ch_body, carry2)

            lax.fori_loop(0, chunk, node_body, carry)
            pltpu.sync_copy(out_v, out_hbm.at[pl.ds(nb, chunk)])
            return carry

        lax.fori_loop(0, nodes_w // chunk, chunk_body, 0)

    return body(y, idx_flat)


def _gather_max(y, idx):
    B, N, H = y.shape
    k = idx.shape[-1]
    hp = 128
    yp = jnp.pad(y[0], ((0, 0), (0, hp - H))) if hp != H else y[0]
    out = _sc_gather_max(yp, idx[0].reshape(-1).astype(jnp.int32), k=k, hp=hp)
    return out[None, :, :H]


def _pairwise_sq(a, b):
    d2 = (jnp.sum(a * a, -1, keepdims=True)
          - 2.0 * jnp.einsum('bnd,bmd->bnm', a, b)
          + jnp.sum(b * b, -1)[:, None, :])
    return jnp.maximum(d2, 0.0)


def _branch(x, idx, W, b):
    C = x.shape[-1]
    Wt, Wb = W[:C], W[C:]
    y = x @ Wb
    c = x @ (Wt - Wb) + b
    return jax.nn.relu(_gather_max(y, idx) + c)


def _block(x, idx1, idx2, p, pre):
    h1 = _branch(x, idx1, p[pre + '_Wa'], p[pre + '_ba'])
    h2 = _branch(x, idx2, p[pre + '_Wb'], p[pre + '_bb'])
    return jax.nn.relu(jnp.concatenate([h1, h2], -1) @ p[pre + '_Wo'] + p[pre + '_bo'])


def _stn15(x, p):
    h = jax.nn.relu(x @ p['stn_W1'] + p['stn_b1'])
    h = jax.nn.relu(h @ p['stn_W2'] + p['stn_b2'])
    h = jax.nn.relu(h @ p['stn_W3'] + p['stn_b3'])
    g = jnp.max(h, axis=1)
    g = jax.nn.relu(g @ p['stn_W4'] + p['stn_b4'])
    g = jax.nn.relu(g @ p['stn_W5'] + p['stn_b5'])
    t = (g @ p['stn_W6'] + p['stn_b6']).reshape(-1, 15, 15) + jnp.eye(15, dtype=x.dtype)[None]
    return jnp.einsum('bnc,bcd->bnd', x, t)


def _res_block(x, p, pre):
    h = jax.nn.relu(x @ p[pre + '_W1'] + p[pre + '_b1'])
    h = h @ p[pre + '_W2'] + p[pre + '_b2']
    return jax.nn.relu(h + x @ p[pre + '_Wp'] + p[pre + '_bp'])


def kernel(x, pos, params):
    p = params
    cd = _pairwise_sq(pos, pos)
    I = lax.top_k(-cd, 512)[1]

    x = _stn15(x, p)

    def feat_idx(xx, lo, hi):
        J = lax.top_k(-_pairwise_sq(xx, xx), hi)[1]
        return J[..., lo:hi]

    h_b1 = _block(x, I[..., 0:32], feat_idx(x, 0, 12), p, 'b1')
    h_f1 = _block(x, I[..., 0:32], I[..., 0:240:20], p, 'f1')
    x1 = jnp.concatenate([h_b1, h_f1], -1)
    h_b2 = _block(x1, I[..., 16:48], feat_idx(x1, 6, 18), p, 'b2')
    h_f2 = _block(x1, I[..., 14:46], I[..., 6:360:36], p, 'f2')
    x2 = jnp.concatenate([h_b2, h_f2], -1)
    h_b3 = _block(x2, I[..., 16:48], feat_idx(x2, 6, 18), p, 'b3')
    h_f3 = _block(x2, I[..., 14:46], I[..., 6:540:54], p, 'f3')
    x3 = jnp.concatenate([h_b3, h_f3], -1)

    h = jnp.concatenate([x1, x2, x3], -1)
    h = jax.nn.relu(h @ p['mlp1_W'] + p['mlp1_b'])
    h = jax.nn.relu(h @ p['mlp2_W'] + p['mlp2_b'])
    w = jax.nn.sigmoid(jnp.max(h, axis=1) @ p['fi_W'] + p['fi_b'])
    h = h * w[:, None, :]
    h = _res_block(h, p, 'r1')
    h = _res_block(h, p, 'r2')
    return h @ p['out_W'] + p['out_b']
